# Initial kernel scaffold; baseline (speedup 1.0000x reference)
import jax
import jax.numpy as jnp
from jax.experimental import pallas as pl


def kernel(feature, edge_index, W, b):
    raise NotImplementedError("write your pallas kernel here")


# speedup vs baseline: 4.8818x; 4.8818x over previous
import functools

import jax
import jax.numpy as jnp
from jax import lax
from jax.experimental import pallas as pl
from jax.experimental.pallas import tpu as pltpu
from jax.experimental.pallas import tpu_sc as plsc

N_NODES = 10000
D = 128
NC = 2
NS = 16
NW = NC * NS
CHUNK = 128
ROWS_PER_TILE_ZERO = 128
N_PAD = 10240
ZCOPIES = N_PAD // (NS * ROWS_PER_TILE_ZERO)
ROWS_PER_TILE = N_PAD // NS


def _sc_aggregate(feature, src3, dst3, cpw):
    mesh = plsc.VectorSubcoreMesh(core_axis_name="c", subcore_axis_name="s")

    @functools.partial(
        pl.kernel,
        mesh=mesh,
        out_type=jax.ShapeDtypeStruct((NC * N_PAD, D), jnp.float32),
        scratch_types=[
            pltpu.VMEM((cpw, CHUNK), jnp.int32),
            pltpu.VMEM((cpw, CHUNK), jnp.int32),
            pltpu.VMEM((CHUNK, D), jnp.float32),
            pltpu.VMEM_SHARED((N_PAD, D), jnp.float32),
            pltpu.SemaphoreType.DMA,
        ],
    )
    def agg(feat_hbm, src_hbm, dst_hbm, out_hbm, src_v, dst_v, rows_v, h_sh, sem):
        c = lax.axis_index("c")
        s = lax.axis_index("s")
        wid = s * NC + c

        zv = jnp.zeros((16,), jnp.float32)

        def zrow(r, _):
            def zlane(j, __):
                rows_v[r, pl.ds(j * 16, 16)] = zv
                return 0
            return lax.fori_loop(0, D // 16, zlane, 0)

        lax.fori_loop(0, ROWS_PER_TILE_ZERO, zrow, 0)

        def zcopy(t, _):
            base = s * ROWS_PER_TILE + t * ROWS_PER_TILE_ZERO
            pltpu.sync_copy(rows_v, h_sh.at[pl.ds(base, ROWS_PER_TILE_ZERO)])
            return 0

        lax.fori_loop(0, ZCOPIES, zcopy, 0)
        plsc.subcore_barrier()

        pltpu.sync_copy(src_hbm.at[wid], src_v)
        pltpu.sync_copy(dst_hbm.at[wid], dst_v)

        def ebody(i, _):
            pltpu.async_copy(feat_hbm.at[src_v.at[i]], rows_v, sem).wait()
            pltpu.sync_copy(rows_v, h_sh.at[dst_v.at[i]], add=True)
            return 0

        lax.fori_loop(0, cpw, ebody, 0)
        plsc.subcore_barrier()

        pltpu.sync_copy(
            h_sh.at[pl.ds(s * ROWS_PER_TILE, ROWS_PER_TILE)],
            out_hbm.at[pl.ds(c * N_PAD + s * ROWS_PER_TILE, ROWS_PER_TILE)],
        )

    return agg(feature, src3, dst3)


def _tc_linear(partials, W, b2):
    nblk = 8
    R = N_PAD // nblk

    def mm(h0_ref, h1_ref, w_ref, b_ref, o_ref):
        h = h0_ref[...] + h1_ref[...]
        acc = lax.dot_general(
            h, w_ref[...], (((1,), (1,)), ((), ())),
            preferred_element_type=jnp.float32,
        )
        o_ref[...] = jnp.maximum(acc + b_ref[...], 0.0)

    return pl.pallas_call(
        mm,
        grid=(nblk,),
        in_specs=[
            pl.BlockSpec((R, D), lambda i: (i, 0)),
            pl.BlockSpec((R, D), lambda i: (i + nblk, 0)),
            pl.BlockSpec((D, D), lambda i: (0, 0)),
            pl.BlockSpec((1, D), lambda i: (0, 0)),
        ],
        out_specs=pl.BlockSpec((R, D), lambda i: (i, 0)),
        out_shape=jax.ShapeDtypeStruct((N_PAD, D), jnp.float32),
    )(partials, partials, W, b2)


def kernel(feature, edge_index, W, b):
    E = edge_index.shape[1]
    cpw = -(-E // (NW * CHUNK))
    e_pad = NW * cpw * CHUNK

    src = edge_index[0].astype(jnp.int32)
    dst = edge_index[1].astype(jnp.int32)
    pad = e_pad - E
    if pad:
        src = jnp.concatenate([src, jnp.zeros((pad,), jnp.int32)])
        dst = jnp.concatenate([dst, jnp.full((pad,), N_PAD - 1, jnp.int32)])
    src3 = src.reshape(NW, cpw, CHUNK)
    dst3 = dst.reshape(NW, cpw, CHUNK)

    partials = _sc_aggregate(feature, src3, dst3, cpw)
    out_pad = _tc_linear(partials, W, b.reshape(1, D))
    return out_pad[:N_NODES]


# speedup vs baseline: 12.8536x; 2.6329x over previous
import functools

import jax
import jax.numpy as jnp
from jax import lax
from jax.experimental import pallas as pl
from jax.experimental.pallas import tpu as pltpu
from jax.experimental.pallas import tpu_sc as plsc

N_NODES = 10000
D = 128
NC = 2
NS = 16
NW = NC * NS
CHUNK = 128
G = 8
N_PAD = 10240
ZROWS = 128
ZCOPIES = N_PAD // (NS * ZROWS)
ROWS_PER_TILE = N_PAD // NS


def _sc_aggregate(feature, src3, dst4, ng):
    mesh = plsc.VectorSubcoreMesh(core_axis_name="c", subcore_axis_name="s")

    @functools.partial(
        pl.kernel,
        mesh=mesh,
        out_type=jax.ShapeDtypeStruct((NC * N_PAD, D), jnp.float32),
        scratch_types=[
            pltpu.VMEM((G * CHUNK,), jnp.int32),
            pltpu.VMEM((G * CHUNK,), jnp.int32),
            pltpu.VMEM((G, CHUNK), jnp.int32),
            pltpu.VMEM((G, CHUNK), jnp.int32),
            pltpu.VMEM((CHUNK, D), jnp.float32),
            pltpu.VMEM((CHUNK, D), jnp.float32),
            pltpu.VMEM_SHARED((N_PAD, D), jnp.float32),
            pltpu.SemaphoreType.DMA,
            pltpu.SemaphoreType.DMA,
            pltpu.SemaphoreType.DMA,
            pltpu.SemaphoreType.DMA,
        ],
    )
    def agg(feat_hbm, src_hbm, dst_hbm, out_hbm,
            src_g0, src_g1, dst_g0, dst_g1, rows_a, rows_b, h_sh,
            sem_a, sem_b, sem_i0, sem_i1):
        c = lax.axis_index("c")
        s = lax.axis_index("s")
        wid = s * NC + c

        src_bufs = (src_g0, src_g1)
        dst_bufs = (dst_g0, dst_g1)
        idx_sems = (sem_i0, sem_i1)
        row_bufs = (rows_a, rows_b)
        row_sems = (sem_a, sem_b)

        zv = jnp.zeros((16,), jnp.float32)

        def zrow(r, _):
            def zlane(j, __):
                rows_a[r, pl.ds(j * 16, 16)] = zv
                return 0
            return lax.fori_loop(0, D // 16, zlane, 0)

        lax.fori_loop(0, ZROWS, zrow, 0)

        def zcopy(t, _):
            base = s * ROWS_PER_TILE + t * ZROWS
            pltpu.sync_copy(rows_a, h_sh.at[pl.ds(base, ZROWS)])
            return 0

        lax.fori_loop(0, ZCOPIES, zcopy, 0)

        def idx_start(g, p):
            pltpu.async_copy(src_hbm.at[wid * ng + g], src_bufs[p], idx_sems[p])
            pltpu.async_copy(dst_hbm.at[wid * ng + g], dst_bufs[p], idx_sems[p])

        def idx_drain(g, p):
            pltpu.make_async_copy(
                src_hbm.at[wid * ng + g], src_bufs[p], idx_sems[p]).wait()
            pltpu.make_async_copy(
                dst_hbm.at[wid * ng + g], dst_bufs[p], idx_sems[p]).wait()

        def gather_start(p_idx, k, p_row):
            pltpu.async_copy(
                feat_hbm.at[src_bufs[p_idx].at[pl.ds(k * CHUNK, CHUNK)]],
                row_bufs[p_row], row_sems[p_row])

        def gather_wait(p_idx, k, p_row):
            pltpu.make_async_copy(
                feat_hbm.at[src_bufs[p_idx].at[pl.ds(k * CHUNK, CHUNK)]],
                row_bufs[p_row], row_sems[p_row]).wait()

        def scatter_add(p_idx, k, p_row):
            pltpu.sync_copy(row_bufs[p_row], h_sh.at[dst_bufs[p_idx].at[k]],
                            add=True)

        def group(g, p, nxt, last=False):
            for k in range(G):
                pr = k % 2
                if k < G - 1:
                    gather_start(p, k + 1, 1 - pr)
                elif not last:
                    idx_drain(*nxt)
                    gather_start(nxt[1], 0, 1 - pr)
                gather_wait(p, k, pr)
                scatter_add(p, k, pr)

        idx_start(0, 0)
        idx_drain(0, 0)
        idx_start(1, 1)
        gather_start(0, 0, 0)
        plsc.subcore_barrier()

        def pair(t, _):
            g0 = 2 * t
            group(g0, 0, (g0 + 1, 1))
            idx_start(g0 + 2, 0)
            group(g0 + 1, 1, (g0 + 2, 0))
            idx_start(g0 + 3, 1)
            return 0

        lax.fori_loop(0, ng // 2 - 1, pair, 0)

        group(ng - 2, 0, (ng - 1, 1))
        group(ng - 1, 1, None, last=True)
        plsc.subcore_barrier()

        pltpu.sync_copy(
            h_sh.at[pl.ds(s * ROWS_PER_TILE, ROWS_PER_TILE)],
            out_hbm.at[pl.ds(c * N_PAD + s * ROWS_PER_TILE, ROWS_PER_TILE)],
        )

    return agg(feature, src3, dst4)


def _tc_linear(partials, W, b2):
    nblk = 8
    R = N_PAD // nblk

    def mm(h0_ref, h1_ref, w_ref, b_ref, o_ref):
        h = h0_ref[...] + h1_ref[...]
        acc = lax.dot_general(
            h, w_ref[...], (((1,), (1,)), ((), ())),
            preferred_element_type=jnp.float32,
        )
        o_ref[...] = jnp.maximum(acc + b_ref[...], 0.0)

    return pl.pallas_call(
        mm,
        grid=(nblk,),
        in_specs=[
            pl.BlockSpec((R, D), lambda i: (i, 0)),
            pl.BlockSpec((R, D), lambda i: (i + nblk, 0)),
            pl.BlockSpec((D, D), lambda i: (0, 0)),
            pl.BlockSpec((1, D), lambda i: (0, 0)),
        ],
        out_specs=pl.BlockSpec((R, D), lambda i: (i, 0)),
        out_shape=jax.ShapeDtypeStruct((N_PAD, D), jnp.float32),
    )(partials, partials, W, b2)


def kernel(feature, edge_index, W, b):
    E = edge_index.shape[1]
    gc = G * CHUNK
    ng = -(-E // (NW * gc))
    ng += ng % 2
    e_pad = NW * ng * gc

    src = edge_index[0].astype(jnp.int32)
    dst = edge_index[1].astype(jnp.int32)
    pad = e_pad - E
    if pad:
        r = jnp.arange(pad, dtype=jnp.int32)
        src = jnp.concatenate([src, r % N_NODES])
        dst = jnp.concatenate([dst, N_NODES + r % (N_PAD - N_NODES)])
    src3 = src.reshape(NW * ng, gc)
    dst4 = dst.reshape(NW * ng, G, CHUNK)

    partials = _sc_aggregate(feature, src3, dst4, ng)
    out_pad = _tc_linear(partials, W, b.reshape(1, D))
    return out_pad[:N_NODES]


# speedup vs baseline: 13.3473x; 1.0384x over previous
import functools

import jax
import jax.numpy as jnp
from jax import lax
from jax.experimental import pallas as pl
from jax.experimental.pallas import tpu as pltpu
from jax.experimental.pallas import tpu_sc as plsc

N_NODES = 10000
D = 128
NC = 2
NS = 16
NW = NC * NS
CHUNK = 128
G = 8
N_PAD = 10240
ZROWS = 128
ZCOPIES = N_PAD // (NS * ZROWS)
ROWS_PER_TILE = N_PAD // NS


def _sc_aggregate(feature, src3, dst4, ng):
    mesh = plsc.VectorSubcoreMesh(core_axis_name="c", subcore_axis_name="s")

    @functools.partial(
        pl.kernel,
        mesh=mesh,
        out_type=(jax.ShapeDtypeStruct((N_PAD, D), jnp.float32),
                  jax.ShapeDtypeStruct((N_PAD, D), jnp.float32)),
        scratch_types=[
            pltpu.VMEM((G * CHUNK,), jnp.int32),
            pltpu.VMEM((G * CHUNK,), jnp.int32),
            pltpu.VMEM((G, CHUNK), jnp.int32),
            pltpu.VMEM((G, CHUNK), jnp.int32),
            pltpu.VMEM((CHUNK, D), jnp.float32),
            pltpu.VMEM((CHUNK, D), jnp.float32),
            pltpu.VMEM_SHARED((N_PAD, D), jnp.float32),
            pltpu.SemaphoreType.DMA,
            pltpu.SemaphoreType.DMA,
            pltpu.SemaphoreType.DMA,
            pltpu.SemaphoreType.DMA,
        ],
    )
    def agg(feat_hbm, src_hbm, dst_hbm, out0_hbm, out1_hbm,
            src_g0, src_g1, dst_g0, dst_g1, rows_a, rows_b, h_sh,
            sem_a, sem_b, sem_i0, sem_i1):
        c = lax.axis_index("c")
        s = lax.axis_index("s")
        wid = s * NC + c

        src_bufs = (src_g0, src_g1)
        dst_bufs = (dst_g0, dst_g1)
        idx_sems = (sem_i0, sem_i1)
        row_bufs = (rows_a, rows_b)
        row_sems = (sem_a, sem_b)

        zv = jnp.zeros((16,), jnp.float32)

        def zrow(r, _):
            def zlane(j, __):
                rows_a[r, pl.ds(j * 16, 16)] = zv
                return 0
            return lax.fori_loop(0, D // 16, zlane, 0)

        lax.fori_loop(0, ZROWS, zrow, 0)

        def zcopy(t, _):
            base = s * ROWS_PER_TILE + t * ZROWS
            pltpu.sync_copy(rows_a, h_sh.at[pl.ds(base, ZROWS)])
            return 0

        lax.fori_loop(0, ZCOPIES, zcopy, 0)

        def idx_start(g, p):
            pltpu.async_copy(src_hbm.at[wid * ng + g], src_bufs[p], idx_sems[p])
            pltpu.async_copy(dst_hbm.at[wid * ng + g], dst_bufs[p], idx_sems[p])

        def idx_drain(g, p):
            pltpu.make_async_copy(
                src_hbm.at[wid * ng + g], src_bufs[p], idx_sems[p]).wait()
            pltpu.make_async_copy(
                dst_hbm.at[wid * ng + g], dst_bufs[p], idx_sems[p]).wait()

        def gather_start(p_idx, k, p_row):
            pltpu.async_copy(
                feat_hbm.at[src_bufs[p_idx].at[pl.ds(k * CHUNK, CHUNK)]],
                row_bufs[p_row], row_sems[p_row])

        def gather_wait(p_idx, k, p_row):
            pltpu.make_async_copy(
                feat_hbm.at[src_bufs[p_idx].at[pl.ds(k * CHUNK, CHUNK)]],
                row_bufs[p_row], row_sems[p_row]).wait()

        def scatter_add(p_idx, k, p_row):
            pltpu.sync_copy(row_bufs[p_row], h_sh.at[dst_bufs[p_idx].at[k]],
                            add=True)

        def group(g, p, nxt, last=False):
            for k in range(G):
                pr = k % 2
                if k < G - 1:
                    gather_start(p, k + 1, 1 - pr)
                elif not last:
                    idx_drain(*nxt)
                    gather_start(nxt[1], 0, 1 - pr)
                gather_wait(p, k, pr)
                scatter_add(p, k, pr)

        idx_start(0, 0)
        idx_drain(0, 0)
        idx_start(1, 1)
        gather_start(0, 0, 0)
        plsc.subcore_barrier()

        def pair(t, _):
            g0 = 2 * t
            group(g0, 0, (g0 + 1, 1))
            idx_start(g0 + 2, 0)
            group(g0 + 1, 1, (g0 + 2, 0))
            idx_start(g0 + 3, 1)
            return 0

        lax.fori_loop(0, ng // 2 - 1, pair, 0)

        group(ng - 2, 0, (ng - 1, 1))
        group(ng - 1, 1, None, last=True)
        plsc.subcore_barrier()

        rows = h_sh.at[pl.ds(s * ROWS_PER_TILE, ROWS_PER_TILE)]

        @pl.when(c == 0)
        def _():
            pltpu.sync_copy(rows, out0_hbm.at[pl.ds(s * ROWS_PER_TILE,
                                                    ROWS_PER_TILE)])

        @pl.when(c == 1)
        def _():
            pltpu.sync_copy(rows, out1_hbm.at[pl.ds(s * ROWS_PER_TILE,
                                                    ROWS_PER_TILE)])

    return agg(feature, src3, dst4)


def _tc_linear(h0, h1, W, b2):
    nblk = 5
    R = N_NODES // nblk

    def mm(h0_ref, h1_ref, w_ref, b_ref, o_ref):
        h = h0_ref[...] + h1_ref[...]
        acc = lax.dot_general(
            h, w_ref[...], (((1,), (1,)), ((), ())),
            preferred_element_type=jnp.float32,
        )
        o_ref[...] = jnp.maximum(acc + b_ref[...], 0.0)

    return pl.pallas_call(
        mm,
        grid=(nblk,),
        in_specs=[
            pl.BlockSpec((R, D), lambda i: (i, 0)),
            pl.BlockSpec((R, D), lambda i: (i, 0)),
            pl.BlockSpec((D, D), lambda i: (0, 0)),
            pl.BlockSpec((1, D), lambda i: (0, 0)),
        ],
        out_specs=pl.BlockSpec((R, D), lambda i: (i, 0)),
        out_shape=jax.ShapeDtypeStruct((N_NODES, D), jnp.float32),
    )(h0, h1, W, b2)


def kernel(feature, edge_index, W, b):
    E = edge_index.shape[1]
    gc = G * CHUNK
    ng = -(-E // (NW * gc))
    ng += ng % 2
    e_pad = NW * ng * gc

    src = edge_index[0].astype(jnp.int32)
    dst = edge_index[1].astype(jnp.int32)
    pad = e_pad - E
    if pad:
        r = jnp.arange(pad, dtype=jnp.int32)
        src = jnp.concatenate([src, r % N_NODES])
        dst = jnp.concatenate([dst, N_NODES + r % (N_PAD - N_NODES)])
    src3 = src.reshape(NW * ng, gc)
    dst4 = dst.reshape(NW * ng, G, CHUNK)

    h0, h1 = _sc_aggregate(feature, src3, dst4, ng)
    return _tc_linear(h0, h1, W, b.reshape(1, D))


# speedup vs baseline: 14.5867x; 1.0929x over previous
import functools

import jax
import jax.numpy as jnp
from jax import lax
from jax.experimental import pallas as pl
from jax.experimental.pallas import tpu as pltpu
from jax.experimental.pallas import tpu_sc as plsc

N_NODES = 10000
D = 128
NC = 2
NS = 16
NW = NC * NS
CHUNK = 64
G = 16
NBUF = 4
N_PAD = 10240
ZROWS = 64
ZCOPIES = N_PAD // (NS * ZROWS)
ROWS_PER_TILE = N_PAD // NS


def _sc_aggregate(feature, src3, dst4, ng):
    mesh = plsc.VectorSubcoreMesh(core_axis_name="c", subcore_axis_name="s")

    @functools.partial(
        pl.kernel,
        mesh=mesh,
        out_type=(jax.ShapeDtypeStruct((N_PAD, D), jnp.float32),
                  jax.ShapeDtypeStruct((N_PAD, D), jnp.float32)),
        scratch_types=[
            pltpu.VMEM((G * CHUNK,), jnp.int32),
            pltpu.VMEM((G * CHUNK,), jnp.int32),
            pltpu.VMEM((G, CHUNK), jnp.int32),
            pltpu.VMEM((G, CHUNK), jnp.int32),
            pltpu.VMEM((CHUNK, D), jnp.float32),
            pltpu.VMEM((CHUNK, D), jnp.float32),
            pltpu.VMEM((CHUNK, D), jnp.float32),
            pltpu.VMEM((CHUNK, D), jnp.float32),
            pltpu.VMEM_SHARED((N_PAD, D), jnp.float32),
            pltpu.SemaphoreType.DMA,
            pltpu.SemaphoreType.DMA,
            pltpu.SemaphoreType.DMA,
            pltpu.SemaphoreType.DMA,
            pltpu.SemaphoreType.DMA,
            pltpu.SemaphoreType.DMA,
        ],
    )
    def agg(feat_hbm, src_hbm, dst_hbm, out0_hbm, out1_hbm,
            src_g0, src_g1, dst_g0, dst_g1, rows_0, rows_1, rows_2, rows_3,
            h_sh, sem_0, sem_1, sem_2, sem_3, sem_i0, sem_i1):
        c = lax.axis_index("c")
        s = lax.axis_index("s")
        wid = s * NC + c

        src_bufs = (src_g0, src_g1)
        dst_bufs = (dst_g0, dst_g1)
        idx_sems = (sem_i0, sem_i1)
        row_bufs = (rows_0, rows_1, rows_2, rows_3)
        row_sems = (sem_0, sem_1, sem_2, sem_3)

        zv = jnp.zeros((16,), jnp.float32)

        def zrow(r, _):
            def zlane(j, __):
                rows_0[r, pl.ds(j * 16, 16)] = zv
                return 0
            return lax.fori_loop(0, D // 16, zlane, 0)

        lax.fori_loop(0, ZROWS, zrow, 0)

        def zcopy(t, _):
            base = s * ROWS_PER_TILE + t * ZROWS
            pltpu.sync_copy(rows_0, h_sh.at[pl.ds(base, ZROWS)])
            return 0

        lax.fori_loop(0, ZCOPIES, zcopy, 0)

        def idx_start(g, p):
            pltpu.async_copy(src_hbm.at[wid * ng + g], src_bufs[p], idx_sems[p])
            pltpu.async_copy(dst_hbm.at[wid * ng + g], dst_bufs[p], idx_sems[p])

        def idx_drain(g, p):
            pltpu.make_async_copy(
                src_hbm.at[wid * ng + g], src_bufs[p], idx_sems[p]).wait()
            pltpu.make_async_copy(
                dst_hbm.at[wid * ng + g], dst_bufs[p], idx_sems[p]).wait()

        def gather_start(p_idx, k, p_row):
            pltpu.async_copy(
                feat_hbm.at[src_bufs[p_idx].at[pl.ds(k * CHUNK, CHUNK)]],
                row_bufs[p_row], row_sems[p_row])

        def gather_wait(p_idx, k, p_row):
            pltpu.make_async_copy(
                feat_hbm.at[src_bufs[p_idx].at[pl.ds(k * CHUNK, CHUNK)]],
                row_bufs[p_row], row_sems[p_row]).wait()

        def scatter_add(p_idx, k, p_row):
            pltpu.sync_copy(row_bufs[p_row], h_sh.at[dst_bufs[p_idx].at[k]],
                            add=True)

        def group(g, p, nxt, last=False):
            for k in range(G):
                j = k + NBUF - 1
                if j < G:
                    gather_start(p, j, j % NBUF)
                elif not last:
                    if j == G:
                        idx_drain(*nxt)
                    gather_start(nxt[1], j - G, j % NBUF)
                gather_wait(p, k, k % NBUF)
                scatter_add(p, k, k % NBUF)

        idx_start(0, 0)
        idx_drain(0, 0)
        idx_start(1, 1)
        for j in range(NBUF - 1):
            gather_start(0, j, j)
        plsc.subcore_barrier()

        def pair(t, _):
            g0 = 2 * t
            group(g0, 0, (g0 + 1, 1))
            idx_start(g0 + 2, 0)
            group(g0 + 1, 1, (g0 + 2, 0))
            idx_start(g0 + 3, 1)
            return 0

        lax.fori_loop(0, ng // 2 - 1, pair, 0)

        group(ng - 2, 0, (ng - 1, 1))
        group(ng - 1, 1, None, last=True)
        plsc.subcore_barrier()

        rows = h_sh.at[pl.ds(s * ROWS_PER_TILE, ROWS_PER_TILE)]

        @pl.when(c == 0)
        def _():
            pltpu.sync_copy(rows, out0_hbm.at[pl.ds(s * ROWS_PER_TILE,
                                                    ROWS_PER_TILE)])

        @pl.when(c == 1)
        def _():
            pltpu.sync_copy(rows, out1_hbm.at[pl.ds(s * ROWS_PER_TILE,
                                                    ROWS_PER_TILE)])

    return agg(feature, src3, dst4)


def _tc_linear(h0, h1, W, b2):
    nblk = 5
    R = N_NODES // nblk

    def mm(h0_ref, h1_ref, w_ref, b_ref, o_ref):
        h = h0_ref[...] + h1_ref[...]
        acc = lax.dot_general(
            h, w_ref[...], (((1,), (1,)), ((), ())),
            preferred_element_type=jnp.float32,
        )
        o_ref[...] = jnp.maximum(acc + b_ref[...], 0.0)

    return pl.pallas_call(
        mm,
        grid=(nblk,),
        in_specs=[
            pl.BlockSpec((R, D), lambda i: (i, 0)),
            pl.BlockSpec((R, D), lambda i: (i, 0)),
            pl.BlockSpec((D, D), lambda i: (0, 0)),
            pl.BlockSpec((1, D), lambda i: (0, 0)),
        ],
        out_specs=pl.BlockSpec((R, D), lambda i: (i, 0)),
        out_shape=jax.ShapeDtypeStruct((N_NODES, D), jnp.float32),
    )(h0, h1, W, b2)


def kernel(feature, edge_index, W, b):
    E = edge_index.shape[1]
    gc = G * CHUNK
    ng = -(-E // (NW * gc))
    ng += ng % 2
    e_pad = NW * ng * gc

    src = edge_index[0].astype(jnp.int32)
    dst = edge_index[1].astype(jnp.int32)
    pad = e_pad - E
    if pad:
        r = jnp.arange(pad, dtype=jnp.int32)
        src = jnp.concatenate([src, r % N_NODES])
        dst = jnp.concatenate([dst, N_NODES + r % (N_PAD - N_NODES)])
    src3 = src.reshape(NW * ng, gc)
    dst4 = dst.reshape(NW * ng, G, CHUNK)

    h0, h1 = _sc_aggregate(feature, src3, dst4, ng)
    return _tc_linear(h0, h1, W, b.reshape(1, D))


# speedup vs baseline: 14.7483x; 1.0111x over previous
import functools

import jax
import jax.numpy as jnp
from jax import lax
from jax.experimental import pallas as pl
from jax.experimental.pallas import tpu as pltpu
from jax.experimental.pallas import tpu_sc as plsc

N_NODES = 10000
D = 128
NC = 2
NS = 16
NW = NC * NS
CHUNK = 64
G = 16
NBUF = 4
N_PAD = 10240
ZROWS = 64
ZCOPIES = N_PAD // (NS * ZROWS)
ROWS_PER_TILE = N_PAD // NS


def _sc_aggregate(feature, src3, dst4, ng):
    mesh = plsc.VectorSubcoreMesh(core_axis_name="c", subcore_axis_name="s")

    @functools.partial(
        pl.kernel,
        mesh=mesh,
        out_type=(jax.ShapeDtypeStruct((N_PAD, D), jnp.float32),
                  jax.ShapeDtypeStruct((N_PAD, D), jnp.float32)),
        scratch_types=[
            pltpu.VMEM((G * CHUNK,), jnp.int32),
            pltpu.VMEM((G * CHUNK,), jnp.int32),
            pltpu.VMEM((G, CHUNK), jnp.int32),
            pltpu.VMEM((G, CHUNK), jnp.int32),
            pltpu.VMEM((CHUNK, D), jnp.float32),
            pltpu.VMEM((CHUNK, D), jnp.float32),
            pltpu.VMEM((CHUNK, D), jnp.float32),
            pltpu.VMEM((CHUNK, D), jnp.float32),
            pltpu.VMEM_SHARED((N_PAD, D), jnp.float32),
            pltpu.SemaphoreType.DMA,
            pltpu.SemaphoreType.DMA,
            pltpu.SemaphoreType.DMA,
            pltpu.SemaphoreType.DMA,
            pltpu.SemaphoreType.DMA,
            pltpu.SemaphoreType.DMA,
            pltpu.SemaphoreType.DMA,
        ],
    )
    def agg(feat_hbm, src_hbm, dst_hbm, out0_hbm, out1_hbm,
            src_g0, src_g1, dst_g0, dst_g1, rows_0, rows_1, rows_2, rows_3,
            h_sh, sem_0, sem_1, sem_2, sem_3, sem_i0, sem_i1, sem_z):
        c = lax.axis_index("c")
        s = lax.axis_index("s")
        wid = s * NC + c

        src_bufs = (src_g0, src_g1)
        dst_bufs = (dst_g0, dst_g1)
        idx_sems = (sem_i0, sem_i1)
        row_bufs = (rows_0, rows_1, rows_2, rows_3)
        row_sems = (sem_0, sem_1, sem_2, sem_3)

        def idx_start(g, p):
            pltpu.async_copy(src_hbm.at[wid * ng + g], src_bufs[p], idx_sems[p])
            pltpu.async_copy(dst_hbm.at[wid * ng + g], dst_bufs[p], idx_sems[p])

        def idx_drain(g, p):
            pltpu.make_async_copy(
                src_hbm.at[wid * ng + g], src_bufs[p], idx_sems[p]).wait()
            pltpu.make_async_copy(
                dst_hbm.at[wid * ng + g], dst_bufs[p], idx_sems[p]).wait()

        def gather_start(p_idx, k, p_row):
            pltpu.async_copy(
                feat_hbm.at[src_bufs[p_idx].at[pl.ds(k * CHUNK, CHUNK)]],
                row_bufs[p_row], row_sems[p_row])

        def gather_wait(p_idx, k, p_row):
            pltpu.make_async_copy(
                feat_hbm.at[src_bufs[p_idx].at[pl.ds(k * CHUNK, CHUNK)]],
                row_bufs[p_row], row_sems[p_row]).wait()

        def scatter_add(p_idx, k, p_row):
            pltpu.sync_copy(row_bufs[p_row], h_sh.at[dst_bufs[p_idx].at[k]],
                            add=True)

        def group(g, p, nxt, last=False):
            for k in range(G):
                j = k + NBUF - 1
                if j < G:
                    gather_start(p, j, j % NBUF)
                elif not last:
                    if j == G:
                        idx_drain(*nxt)
                    gather_start(nxt[1], j - G, j % NBUF)
                gather_wait(p, k, k % NBUF)
                scatter_add(p, k, k % NBUF)

        idx_start(0, 0)
        idx_start(1, 1)

        zv = jnp.zeros((16,), jnp.float32)

        def zrow(r, _):
            def zlane(j, __):
                rows_0[r, pl.ds(j * 16, 16)] = zv
                return 0
            return lax.fori_loop(0, D // 16, zlane, 0)

        lax.fori_loop(0, ZROWS, zrow, 0)

        def zcopy(t, _):
            base = s * ROWS_PER_TILE + t * ZROWS
            pltpu.async_copy(rows_0, h_sh.at[pl.ds(base, ZROWS)], sem_z)
            return 0

        lax.fori_loop(0, ZCOPIES, zcopy, 0)

        def zdrain(t, _):
            base = s * ROWS_PER_TILE + t * ZROWS
            pltpu.make_async_copy(rows_0, h_sh.at[pl.ds(base, ZROWS)],
                                  sem_z).wait()
            return 0

        lax.fori_loop(0, ZCOPIES, zdrain, 0)

        idx_drain(0, 0)
        for j in range(NBUF - 1):
            gather_start(0, j, j)
        plsc.subcore_barrier()

        def pair(t, _):
            g0 = 2 * t
            group(g0, 0, (g0 + 1, 1))
            idx_start(g0 + 2, 0)
            group(g0 + 1, 1, (g0 + 2, 0))
            idx_start(g0 + 3, 1)
            return 0

        lax.fori_loop(0, ng // 2 - 1, pair, 0)

        group(ng - 2, 0, (ng - 1, 1))
        group(ng - 1, 1, None, last=True)
        plsc.subcore_barrier()

        rows = h_sh.at[pl.ds(s * ROWS_PER_TILE, ROWS_PER_TILE)]

        @pl.when(c == 0)
        def _():
            pltpu.sync_copy(rows, out0_hbm.at[pl.ds(s * ROWS_PER_TILE,
                                                    ROWS_PER_TILE)])

        @pl.when(c == 1)
        def _():
            pltpu.sync_copy(rows, out1_hbm.at[pl.ds(s * ROWS_PER_TILE,
                                                    ROWS_PER_TILE)])

    return agg(feature, src3, dst4)


def _tc_linear(h0, h1, W, b2):
    nblk = 5
    R = N_NODES // nblk

    def mm(h0_ref, h1_ref, w_ref, b_ref, o_ref):
        h = h0_ref[...] + h1_ref[...]
        acc = lax.dot_general(
            h, w_ref[...], (((1,), (1,)), ((), ())),
            preferred_element_type=jnp.float32,
        )
        o_ref[...] = jnp.maximum(acc + b_ref[...], 0.0)

    return pl.pallas_call(
        mm,
        grid=(nblk,),
        in_specs=[
            pl.BlockSpec((R, D), lambda i: (i, 0)),
            pl.BlockSpec((R, D), lambda i: (i, 0)),
            pl.BlockSpec((D, D), lambda i: (0, 0)),
            pl.BlockSpec((1, D), lambda i: (0, 0)),
        ],
        out_specs=pl.BlockSpec((R, D), lambda i: (i, 0)),
        out_shape=jax.ShapeDtypeStruct((N_NODES, D), jnp.float32),
    )(h0, h1, W, b2)


def kernel(feature, edge_index, W, b):
    E = edge_index.shape[1]
    gc = G * CHUNK
    ng = -(-E // (NW * gc))
    ng += ng % 2
    e_pad = NW * ng * gc

    src = edge_index[0].astype(jnp.int32)
    dst = edge_index[1].astype(jnp.int32)
    pad = e_pad - E
    if pad:
        r = jnp.arange(pad, dtype=jnp.int32)
        src = jnp.concatenate([src, r % N_NODES])
        dst = jnp.concatenate([dst, N_NODES + r % (N_PAD - N_NODES)])
    src3 = src.reshape(NW * ng, gc)
    dst4 = dst.reshape(NW * ng, G, CHUNK)

    h0, h1 = _sc_aggregate(feature, src3, dst4, ng)
    return _tc_linear(h0, h1, W, b.reshape(1, D))
